# Initial kernel scaffold; baseline (speedup 1.0000x reference)
import jax
import jax.numpy as jnp
from jax.experimental import pallas as pl


def kernel(x, edge_index, params):
    raise NotImplementedError("write your pallas kernel here")


# speedup vs baseline: 1.0239x; 1.0239x over previous
import functools

import jax
import jax.numpy as jnp
from jax import lax
from jax.experimental import pallas as pl
from jax.experimental.pallas import tpu as pltpu

N = 4096
E = 131072
H = 128
OUT = 64
LAYERS = 6
HEADS = 4
DH = 64
D2 = 2 * H
EPS = 1e-5


def _ln(v, g, b):
    m = jnp.mean(v, axis=-1, keepdims=True)
    var = jnp.mean((v - m) ** 2, axis=-1, keepdims=True)
    return (v - m) * lax.rsqrt(var + EPS) * g + b


def _sage_body(do_ln, agg_ref, g_ref, inv_ref, wl_ref, bl_ref, wr_ref,
               lng_ref, lnb_ref, out_ref):
    mean = agg_ref[...] * inv_ref[...]
    h = jnp.dot(mean, wl_ref[...], preferred_element_type=jnp.float32)
    h += jnp.dot(g_ref[...], wr_ref[...], preferred_element_type=jnp.float32)
    h += bl_ref[...]
    if do_ln:
        h = _ln(h, lng_ref[...], lnb_ref[...])
    out_ref[...] = jnp.maximum(h, 0.0) + g_ref[...]


def _sage_update(agg, g, inv, wl, bl, wr, lng, lnb, do_ln):
    BS = 1024
    grid = (N // BS,)
    return pl.pallas_call(
        functools.partial(_sage_body, do_ln),
        grid=grid,
        in_specs=[
            pl.BlockSpec((BS, H), lambda i: (i, 0)),
            pl.BlockSpec((BS, H), lambda i: (i, 0)),
            pl.BlockSpec((BS, 1), lambda i: (i, 0)),
            pl.BlockSpec((H, H), lambda i: (0, 0)),
            pl.BlockSpec((1, H), lambda i: (0, 0)),
            pl.BlockSpec((H, H), lambda i: (0, 0)),
            pl.BlockSpec((1, H), lambda i: (0, 0)),
            pl.BlockSpec((1, H), lambda i: (0, 0)),
        ],
        out_specs=pl.BlockSpec((BS, H), lambda i: (i, 0)),
        out_shape=jax.ShapeDtypeStruct((N, H), jnp.float32),
    )(agg, g, inv, wl, bl.reshape(1, H), wr, lng.reshape(1, H), lnb.reshape(1, H))


def _cnn_body(g_ref, w_ref, b_ref, lng_ref, lnb_ref, fused_ref):
    c = g_ref[...]
    for l in range(3):
        cm = jnp.concatenate([jnp.zeros((1, H), jnp.float32), c[:-1]], axis=0)
        cp = jnp.concatenate([c[1:], jnp.zeros((1, H), jnp.float32)], axis=0)
        h = jnp.dot(cm, w_ref[l, 0], preferred_element_type=jnp.float32)
        h += jnp.dot(c, w_ref[l, 1], preferred_element_type=jnp.float32)
        h += jnp.dot(cp, w_ref[l, 2], preferred_element_type=jnp.float32)
        h += b_ref[l]
        h = jnp.maximum(h, 0.0)
        c = _ln(h, lng_ref[l], lnb_ref[l])
    fused_ref[...] = jnp.concatenate([g_ref[...], c], axis=1)


def _cnn(g, w, b, lng, lnb):
    return pl.pallas_call(
        _cnn_body,
        out_shape=jax.ShapeDtypeStruct((N, D2), jnp.float32),
    )(g, w, b, lng, lnb)


def _qkv_body(f_ref, w_ref, b_ref, q_ref, k_ref, v_ref):
    f = f_ref[...]
    for h in range(HEADS):
        q_ref[h] = (jnp.dot(f, w_ref[0, h], preferred_element_type=jnp.float32)
                    + b_ref[0, h])
        k_ref[h] = (jnp.dot(f, w_ref[1, h], preferred_element_type=jnp.float32)
                    + b_ref[1, h])
        v_ref[h] = (jnp.dot(f, w_ref[2, h], preferred_element_type=jnp.float32)
                    + b_ref[2, h])


def _qkv(fused, w3, b3):
    BS = 1024
    hspec = pl.BlockSpec((HEADS, BS, DH), lambda i: (0, i, 0))
    shp = jax.ShapeDtypeStruct((HEADS, N, DH), jnp.float32)
    return pl.pallas_call(
        _qkv_body,
        grid=(N // BS,),
        in_specs=[
            pl.BlockSpec((BS, D2), lambda i: (i, 0)),
            pl.BlockSpec((3, HEADS, D2, DH), lambda i: (0, 0, 0, 0)),
            pl.BlockSpec((3, HEADS, 1, DH), lambda i: (0, 0, 0, 0)),
        ],
        out_specs=[hspec, hspec, hspec],
        out_shape=[shp, shp, shp],
    )(fused, w3, b3)


def _attn_body(q_ref, k_ref, v_ref, o_ref):
    q = q_ref[0]
    k = k_ref[0]
    s = lax.dot_general(q, k, (((1,), (1,)), ((), ())),
                        preferred_element_type=jnp.float32)
    s *= 1.0 / (float(DH) ** 0.5)
    m = jnp.max(s, axis=-1, keepdims=True)
    p = jnp.exp(s - m)
    l = jnp.sum(p, axis=-1, keepdims=True)
    p = p / l
    o_ref[0] = jnp.dot(p, v_ref[0], preferred_element_type=jnp.float32)


def _attention(q, k, v):
    BQ = 512
    grid = (HEADS, N // BQ)
    return pl.pallas_call(
        _attn_body,
        grid=grid,
        in_specs=[
            pl.BlockSpec((1, BQ, DH), lambda h, i: (h, i, 0)),
            pl.BlockSpec((1, N, DH), lambda h, i: (h, 0, 0)),
            pl.BlockSpec((1, N, DH), lambda h, i: (h, 0, 0)),
        ],
        out_specs=pl.BlockSpec((1, BQ, DH), lambda h, i: (h, i, 0)),
        out_shape=jax.ShapeDtypeStruct((HEADS, N, DH), jnp.float32),
    )(q, k, v)


def _final_body(o_ref, wT_ref, b_ref, lng_ref, lnb_ref, fw_ref, fb_ref, out_ref):
    t = jnp.dot(o_ref[0], wT_ref[0], preferred_element_type=jnp.float32)
    for h in range(1, HEADS):
        t += jnp.dot(o_ref[h], wT_ref[h], preferred_element_type=jnp.float32)
    t += b_ref[...]
    t = _ln(t, lng_ref[...], lnb_ref[...])
    out_ref[...] = (
        jnp.dot(t, fw_ref[...], preferred_element_type=jnp.float32) + fb_ref[...]
    )


def _final(o, wT, b, lng, lnb, fw, fb):
    BS = 1024
    return pl.pallas_call(
        _final_body,
        grid=(N // BS,),
        in_specs=[
            pl.BlockSpec((HEADS, BS, DH), lambda i: (0, i, 0)),
            pl.BlockSpec((HEADS, DH, D2), lambda i: (0, 0, 0)),
            pl.BlockSpec((1, D2), lambda i: (0, 0)),
            pl.BlockSpec((1, D2), lambda i: (0, 0)),
            pl.BlockSpec((1, D2), lambda i: (0, 0)),
            pl.BlockSpec((D2, OUT), lambda i: (0, 0)),
            pl.BlockSpec((1, OUT), lambda i: (0, 0)),
        ],
        out_specs=pl.BlockSpec((BS, OUT), lambda i: (i, 0)),
        out_shape=jax.ShapeDtypeStruct((N, OUT), jnp.float32),
    )(o, wT, b.reshape(1, D2), lng.reshape(1, D2), lnb.reshape(1, D2),
      fw, fb.reshape(1, OUT))


def kernel(x, edge_index, params):
    src = edge_index[0]
    dst = edge_index[1]
    deg = jax.ops.segment_sum(jnp.ones((E,), jnp.float32), dst, num_segments=N)
    inv = (1.0 / jnp.maximum(deg, 1.0)).reshape(N, 1)

    g = x
    for l in range(LAYERS):
        msg = jnp.take(g, src, axis=0)
        agg = jax.ops.segment_sum(msg, dst, num_segments=N)
        sp = params["sage"][l]
        if l < LAYERS - 1:
            ln = params["gnn_ln"][l]
            g = _sage_update(agg, g, inv, sp["Wl"], sp["bl"], sp["Wr"],
                             ln["g"], ln["b"], True)
        else:
            z = jnp.zeros((H,), jnp.float32)
            g = _sage_update(agg, g, inv, sp["Wl"], sp["bl"], sp["Wr"],
                             z, z, False)

    w = jnp.stack([jnp.transpose(params["conv"][l]["w"], (2, 1, 0))
                   for l in range(3)])
    b = jnp.stack([params["conv"][l]["b"] for l in range(3)]).reshape(3, 1, H)
    lng = jnp.stack([params["cnn_ln"][l]["g"] for l in range(3)]).reshape(3, 1, H)
    lnb = jnp.stack([params["cnn_ln"][l]["b"] for l in range(3)]).reshape(3, 1, H)
    fused = _cnn(g, w, b, lng, lnb)

    a = params["attn"]
    w3 = a["in_w"].reshape(3, HEADS, DH, D2).transpose(0, 1, 3, 2)
    b3 = a["in_b"].reshape(3, HEADS, 1, DH)
    q, k, v = _qkv(fused, w3, b3)
    o = _attention(q, k, v)
    aln = params["attn_ln"]
    fu = params["fusion"]
    woT = a["out_w"].reshape(D2, HEADS, DH).transpose(1, 2, 0)
    return _final(o, woT, a["out_b"], aln["g"], aln["b"],
                  fu["W"], fu["b"])


# speedup vs baseline: 9.4496x; 9.2293x over previous
import functools

import jax
import jax.numpy as jnp
from jax import lax
from jax.experimental import pallas as pl
from jax.experimental.pallas import tpu as pltpu
from jax.experimental.pallas import tpu_sc as plsc

N = 4096
E = 131072
H = 128
OUT = 64
LAYERS = 6
HEADS = 4
DH = 64
D2 = 2 * H
EPS = 1e-5

NC = 2
NS = 16
NW = NC * NS
EPW = E // NW
CH = 128
NCH = EPW // CH
RPS = N // NS

def _sc_mesh():
    return plsc.VectorSubcoreMesh(
        core_axis_name="c", subcore_axis_name="s",
        num_cores=NC, num_subcores=NS)


NBUF = 4


def _sc_segsum_body(g_hbm, src_hbm, dst_hbm, z_hbm, out_hbm,
                    srcv, dstv, r0, r1, r2, r3, acc, s0, s1, s2, s3):
    rows = (r0, r1, r2, r3)
    sems = (s0, s1, s2, s3)
    c = lax.axis_index("c")
    s = lax.axis_index("s")
    wid = s * NC + c
    pltpu.sync_copy(z_hbm, acc.at[pl.ds(s * RPS, RPS)])
    pltpu.sync_copy(src_hbm.at[wid], srcv)
    pltpu.sync_copy(dst_hbm.at[wid], dstv)
    plsc.subcore_barrier()

    for b in range(NBUF - 1):
        pltpu.async_copy(g_hbm.at[srcv.at[b]], rows[b], sems[b])

    def blk(jo, carry):
        base = jo * NBUF
        for b in range(NBUF):
            j = base + b

            @pl.when(j + NBUF - 1 < NCH)
            def _():
                pltpu.async_copy(g_hbm.at[srcv.at[j + NBUF - 1]],
                                 rows[(b + NBUF - 1) % NBUF],
                                 sems[(b + NBUF - 1) % NBUF])

            pltpu.make_async_copy(g_hbm.at[srcv.at[j]], rows[b],
                                  sems[b]).wait()
            pltpu.sync_copy(rows[b], acc.at[dstv.at[j]], add=True)
        return carry

    lax.fori_loop(0, NCH // NBUF, blk, 0)
    plsc.subcore_barrier()
    pltpu.sync_copy(acc.at[pl.ds(s * RPS, RPS)],
                    out_hbm.at[c, pl.ds(s * RPS, RPS)])


def _sc_segsum(g, src3, dst3, zrows):
    k = pl.kernel(
        _sc_segsum_body,
        out_type=jax.ShapeDtypeStruct((NC, N, H), jnp.float32),
        mesh=_sc_mesh(),
        scratch_types=[
            pltpu.VMEM((NCH, CH), jnp.int32),
            pltpu.VMEM((NCH, CH), jnp.int32),
            pltpu.VMEM((CH, H), jnp.float32),
            pltpu.VMEM((CH, H), jnp.float32),
            pltpu.VMEM((CH, H), jnp.float32),
            pltpu.VMEM((CH, H), jnp.float32),
            pltpu.VMEM_SHARED((N, H), jnp.float32),
            pltpu.SemaphoreType.DMA,
            pltpu.SemaphoreType.DMA,
            pltpu.SemaphoreType.DMA,
            pltpu.SemaphoreType.DMA,
        ],
    )
    return k(g, src3, dst3, zrows)


def _sc_degree_body(dst_hbm, ones_hbm, z_hbm, out_hbm, dstv, onesb, dacc, sem):
    c = lax.axis_index("c")
    s = lax.axis_index("s")
    wid = s * NC + c
    pltpu.sync_copy(z_hbm, dacc.at[pl.ds(s * RPS, RPS)])
    pltpu.sync_copy(ones_hbm, onesb)
    pltpu.sync_copy(dst_hbm.at[wid], dstv)
    plsc.subcore_barrier()

    def chunk(j, carry):
        pltpu.sync_copy(onesb, dacc.at[dstv.at[j]], add=True)
        return carry

    lax.fori_loop(0, NCH, chunk, 0)
    plsc.subcore_barrier()
    pltpu.sync_copy(dacc.at[pl.ds(s * RPS, RPS)],
                    out_hbm.at[c, pl.ds(s * RPS, RPS)])


def _sc_degree(dst3, ones_rows, zrows):
    k = pl.kernel(
        _sc_degree_body,
        out_type=jax.ShapeDtypeStruct((NC, N, H), jnp.float32),
        mesh=_sc_mesh(),
        scratch_types=[
            pltpu.VMEM((NCH, CH), jnp.int32),
            pltpu.VMEM((CH, H), jnp.float32),
            pltpu.VMEM_SHARED((N, H), jnp.float32),
            pltpu.SemaphoreType.DMA,
        ],
    )
    return k(dst3, ones_rows, zrows)


def _ln(v, g, b):
    m = jnp.mean(v, axis=-1, keepdims=True)
    var = jnp.mean((v - m) ** 2, axis=-1, keepdims=True)
    return (v - m) * lax.rsqrt(var + EPS) * g + b


def _sage_body(do_ln, agg_ref, g_ref, inv_ref, wl_ref, bl_ref, wr_ref,
               lng_ref, lnb_ref, out_ref):
    denom = jnp.maximum(inv_ref[0] + inv_ref[1], 1.0)
    mean = (agg_ref[0] + agg_ref[1]) / denom
    h = jnp.dot(mean, wl_ref[...], preferred_element_type=jnp.float32)
    h += jnp.dot(g_ref[...], wr_ref[...], preferred_element_type=jnp.float32)
    h += bl_ref[...]
    if do_ln:
        h = _ln(h, lng_ref[...], lnb_ref[...])
    out_ref[...] = jnp.maximum(h, 0.0) + g_ref[...]


def _sage_update(agg, g, inv, wl, bl, wr, lng, lnb, do_ln):
    BS = 2048
    grid = (N // BS,)
    return pl.pallas_call(
        functools.partial(_sage_body, do_ln),
        grid=grid,
        in_specs=[
            pl.BlockSpec((NC, BS, H), lambda i: (0, i, 0)),
            pl.BlockSpec((BS, H), lambda i: (i, 0)),
            pl.BlockSpec((NC, BS, H), lambda i: (0, i, 0)),
            pl.BlockSpec((H, H), lambda i: (0, 0)),
            pl.BlockSpec((1, H), lambda i: (0, 0)),
            pl.BlockSpec((H, H), lambda i: (0, 0)),
            pl.BlockSpec((1, H), lambda i: (0, 0)),
            pl.BlockSpec((1, H), lambda i: (0, 0)),
        ],
        out_specs=pl.BlockSpec((BS, H), lambda i: (i, 0)),
        out_shape=jax.ShapeDtypeStruct((N, H), jnp.float32),
    )(agg, g, inv, wl, bl.reshape(1, H), wr, lng.reshape(1, H),
      lnb.reshape(1, H))


BQ = 1024


def _tail_body(g_ref, cw_ref, cb_ref, clng_ref, clnb_ref, w3_ref, b3_ref,
               wo_ref, bo_ref, alng_ref, alnb_ref, fw_ref, fb_ref, out_ref,
               fused_scr, k_scr, v_scr, o_scr):
    hh = pl.program_id(0)
    qi = pl.program_id(1)

    @pl.when(jnp.logical_and(hh == 0, qi == 0))
    def _():
        c = g_ref[...]
        for l in range(3):
            cm = jnp.concatenate([jnp.zeros((1, H), jnp.float32), c[:-1]],
                                 axis=0)
            cp = jnp.concatenate([c[1:], jnp.zeros((1, H), jnp.float32)],
                                 axis=0)
            t = jnp.dot(cm, cw_ref[l, 0], preferred_element_type=jnp.float32)
            t += jnp.dot(c, cw_ref[l, 1], preferred_element_type=jnp.float32)
            t += jnp.dot(cp, cw_ref[l, 2], preferred_element_type=jnp.float32)
            t += cb_ref[l]
            t = jnp.maximum(t, 0.0)
            c = _ln(t, clng_ref[l], clnb_ref[l])
        fused_scr[...] = jnp.concatenate([g_ref[...], c], axis=1)

    @pl.when(qi == 0)
    def _():
        f = fused_scr[...]
        k_scr[...] = lax.dot_general(
            f, w3_ref[1, 0], (((1,), (1,)), ((), ())),
            preferred_element_type=jnp.float32).astype(jnp.bfloat16)
        v_scr[...] = lax.dot_general(
            f, w3_ref[2, 0], (((1,), (1,)), ((), ())),
            preferred_element_type=jnp.float32).astype(jnp.bfloat16)

    fq = fused_scr[pl.ds(qi * BQ, BQ), :]
    q = (lax.dot_general(fq, w3_ref[0, 0], (((1,), (1,)), ((), ())),
                         preferred_element_type=jnp.float32)
         + b3_ref[0, 0, 0]) * (1.0 / float(DH) ** 0.5)
    s = lax.dot_general(q.astype(jnp.bfloat16), k_scr[...],
                        (((1,), (1,)), ((), ())),
                        preferred_element_type=jnp.float32)
    m = jnp.max(s, axis=-1, keepdims=True)
    p = jnp.exp(s - m).astype(jnp.bfloat16)
    l = jnp.sum(p.astype(jnp.float32), axis=-1, keepdims=True)
    o = jnp.dot(p, v_scr[...], preferred_element_type=jnp.float32)
    o_scr[hh, pl.ds(qi * BQ, BQ)] = o / l

    @pl.when(jnp.logical_and(hh == HEADS - 1, qi == N // BQ - 1))
    def _():
        t = lax.dot_general(o_scr[0], wo_ref[0], (((1,), (1,)), ((), ())),
                            preferred_element_type=jnp.float32)
        for h2 in range(1, HEADS):
            t += lax.dot_general(o_scr[h2], wo_ref[h2],
                                 (((1,), (1,)), ((), ())),
                                 preferred_element_type=jnp.float32)
        t += bo_ref[...]
        t = _ln(t, alng_ref[...], alnb_ref[...])
        out_ref[...] = (
            jnp.dot(t, fw_ref[...], preferred_element_type=jnp.float32)
            + fb_ref[...]
        )


def _dense_tail(g, cw, cb, clng, clnb, w3r, b3, wor, bo, alng, alnb, fw, fb):
    grid = (HEADS, N // BQ)
    call = pl.pallas_call(
        _tail_body,
        grid=grid,
        in_specs=[
            pl.BlockSpec((N, H), lambda h, i: (0, 0)),
            pl.BlockSpec((3, 3, H, H), lambda h, i: (0, 0, 0, 0)),
            pl.BlockSpec((3, 1, H), lambda h, i: (0, 0, 0)),
            pl.BlockSpec((3, 1, H), lambda h, i: (0, 0, 0)),
            pl.BlockSpec((3, 1, H), lambda h, i: (0, 0, 0)),
            pl.BlockSpec((3, 1, DH, D2), lambda h, i: (0, h, 0, 0)),
            pl.BlockSpec((3, 1, 1, DH), lambda h, i: (0, h, 0, 0)),
            pl.BlockSpec((HEADS, D2, DH), lambda h, i: (0, 0, 0)),
            pl.BlockSpec((1, D2), lambda h, i: (0, 0)),
            pl.BlockSpec((1, D2), lambda h, i: (0, 0)),
            pl.BlockSpec((1, D2), lambda h, i: (0, 0)),
            pl.BlockSpec((D2, OUT), lambda h, i: (0, 0)),
            pl.BlockSpec((1, OUT), lambda h, i: (0, 0)),
        ],
        out_specs=pl.BlockSpec((N, OUT), lambda h, i: (0, 0)),
        out_shape=jax.ShapeDtypeStruct((N, OUT), jnp.float32),
        scratch_shapes=[
            pltpu.VMEM((N, D2), jnp.float32),
            pltpu.VMEM((N, DH), jnp.bfloat16),
            pltpu.VMEM((N, DH), jnp.bfloat16),
            pltpu.VMEM((HEADS, N, DH), jnp.float32),
        ],
    )
    return call(g, cw, cb, clng, clnb, w3r, b3, wor, bo, alng, alnb, fw, fb)


def kernel(x, edge_index, params):
    src3 = edge_index[0].reshape(NW, NCH, CH)
    dst3 = edge_index[1].reshape(NW, NCH, CH)
    zrows = jnp.zeros((RPS, H), jnp.float32)
    ones_rows = jnp.ones((CH, H), jnp.float32)

    dpart = _sc_degree(dst3, ones_rows, zrows)
    g = x
    for l in range(LAYERS):
        sp = params["sage"][l]
        agg = _sc_segsum(g, src3, dst3, zrows)
        if l < LAYERS - 1:
            ln = params["gnn_ln"][l]
            g = _sage_update(agg, g, dpart, sp["Wl"], sp["bl"], sp["Wr"],
                             ln["g"], ln["b"], True)
        else:
            z = jnp.zeros((H,), jnp.float32)
            g = _sage_update(agg, g, dpart, sp["Wl"], sp["bl"], sp["Wr"],
                             z, z, False)

    cw = jnp.stack([jnp.transpose(params["conv"][l]["w"], (2, 1, 0))
                    for l in range(3)])
    cb = jnp.stack([params["conv"][l]["b"] for l in range(3)]).reshape(3, 1, H)
    clng = jnp.stack([params["cnn_ln"][l]["g"]
                      for l in range(3)]).reshape(3, 1, H)
    clnb = jnp.stack([params["cnn_ln"][l]["b"]
                      for l in range(3)]).reshape(3, 1, H)

    a = params["attn"]
    w3r = a["in_w"].reshape(3, HEADS, DH, D2)
    b3 = a["in_b"].reshape(3, HEADS, 1, DH)
    wor = a["out_w"].reshape(D2, HEADS, DH).transpose(1, 0, 2)
    aln = params["attn_ln"]
    fu = params["fusion"]
    bo_eff = a["out_b"] + a["in_b"][2 * D2:] @ a["out_w"].T
    return _dense_tail(g, cw, cb, clng, clnb, w3r, b3,
                       wor, bo_eff.reshape(1, D2),
                       aln["g"].reshape(1, D2), aln["b"].reshape(1, D2),
                       fu["W"], fu["b"].reshape(1, OUT))
